# baseline (speedup 1.0000x reference)
import functools

import jax
import jax.numpy as jnp
from jax import lax
from jax.experimental import pallas as pl
from jax.experimental.pallas import tpu as pltpu
from jax.experimental.pallas import tpu_sc as plsc

B = 4096
H = 200
D = 64
TOTAL = 409600
HD = H * D

NC = 2
NS = 16
NW = NC * NS
SPW = B // NW
WIN = 208

_sc_mesh = plsc.VectorSubcoreMesh(core_axis_name="c", subcore_axis_name="s")


@functools.partial(
    pl.kernel,
    mesh=_sc_mesh,
    out_type=jax.ShapeDtypeStruct((B * H, D), jnp.float32),
    scratch_types=[
        pltpu.VMEM((SPW + 16,), jnp.int32),
        pltpu.VMEM((WIN,), jnp.int32),
        pltpu.VMEM((128,), jnp.int32),
        pltpu.VMEM((80,), jnp.int32),
        pltpu.VMEM((128, D), jnp.float32),
        pltpu.VMEM((80, D), jnp.float32),
        pltpu.SemaphoreType.DMA,
    ],
    compiler_params=pltpu.CompilerParams(needs_layout_passes=False,
                                         use_tc_tiling_on_sc=False),
)
def _sc_gather(values_hbm, starts_hbm, table_hbm, out_hbm,
               starts_v, win_v, ids_a, ids_b, rows_a, rows_b, sem):
    wid = lax.axis_index("s") * NC + lax.axis_index("c")
    base = wid * SPW
    pltpu.sync_copy(starts_hbm.at[pl.ds(base, SPW)], starts_v.at[pl.ds(0, SPW)])

    GRP = 8

    def body(g, carry):
        svec = starts_v[pl.ds(g * GRP, 16)]
        for k in range(GRP):
            start = svec[k]
            base0 = pl.multiple_of(jnp.minimum(start & ~7, TOTAL - WIN), 8)
            sh = start - base0
            lim = jnp.int32(TOTAL - 1) - base0
            pltpu.sync_copy(values_hbm.at[pl.ds(base0, WIN)], win_v)
            for j in range(13):
                kvec = lax.iota(jnp.int32, 16) + (sh + j * 16)
                q = jnp.minimum(kvec, lim)
                ids = plsc.load_gather(win_v, [q])
                if j < 8:
                    ids_a[pl.ds(j * 16, 16)] = ids
                else:
                    ids_b[pl.ds(j * 16 - 128, 16)] = ids
            c1 = pltpu.async_copy(table_hbm.at[ids_a], rows_a, sem)
            c2 = pltpu.async_copy(table_hbm.at[ids_b], rows_b, sem)
            c1.wait()
            c2.wait()
            orow = (base + g * GRP + k) * H
            pltpu.sync_copy(rows_a, out_hbm.at[pl.ds(orow, 128)])
            pltpu.sync_copy(rows_b.at[pl.ds(0, 72)],
                            out_hbm.at[pl.ds(orow + 128, 72)])
        return carry

    lax.fori_loop(0, SPW // GRP, body, 0)


def _ln_body(len_ref, x_ref, pos_ref, w_ref, b_ref, o_ref):
    x = x_ref[...]
    col = lax.broadcasted_iota(jnp.int32, x.shape, 1)
    h = col >> 6
    valid = h < len_ref[...]
    xm = jnp.where(valid, x, 0.0) + pos_ref[...]
    mean = jnp.mean(xm, axis=1, keepdims=True)
    xc = xm - mean
    var = jnp.mean(xc * xc, axis=1, keepdims=True)
    o_ref[...] = xc * lax.rsqrt(var + 1e-5) * w_ref[...] + b_ref[...]


def kernel(values, offsets, table, pos_enc, ln_weight, ln_bias):
    values = values.astype(jnp.int32)
    starts = offsets[:-1].astype(jnp.int32)
    lengths = (offsets[1:] - offsets[:-1]).astype(jnp.int32)
    dense = _sc_gather(values, starts, table)
    x = dense.reshape(B, HD)

    BB = 64
    out = pl.pallas_call(
        _ln_body,
        grid=(B // BB,),
        in_specs=[
            pl.BlockSpec((BB, 1), lambda i: (i, 0)),
            pl.BlockSpec((BB, HD), lambda i: (i, 0)),
            pl.BlockSpec((1, HD), lambda i: (0, 0)),
            pl.BlockSpec((1, HD), lambda i: (0, 0)),
            pl.BlockSpec((1, HD), lambda i: (0, 0)),
        ],
        out_specs=pl.BlockSpec((BB, HD), lambda i: (i, 0)),
        out_shape=jax.ShapeDtypeStruct((B, HD), jnp.float32),
    )(
        lengths.reshape(B, 1),
        x,
        pos_enc.reshape(1, HD),
        ln_weight.reshape(1, HD),
        ln_bias.reshape(1, HD),
    )
    return out.reshape(B, H, D)
